# baseline (speedup 1.0000x reference)
import functools

import jax
import jax.numpy as jnp
from jax import lax
from jax.experimental import pallas as pl
from jax.experimental.pallas import tpu as pltpu
from jax.experimental.pallas import tpu_sc as plsc

N, E, D = 10000, 320000, 128

_INFO = plsc.get_sparse_core_info()
NC, NS, L = _INFO.num_cores, _INFO.num_subcores, _INFO.num_lanes
NW = NC * NS

CHUNK = 64
GROUP = 8
CHUNKS_C0 = 264
CHUNKS_C1 = 56
CORE1_BASE = NS * CHUNKS_C0
TOTAL_CHUNKS = NS * (CHUNKS_C0 + CHUNKS_C1)
E_PAD = TOTAL_CHUNKS * CHUNK
NBUF = 2

N_PAD = 10008
ROW_SPLIT = 632
ROW_LAST = N_PAD - (NS - 1) * ROW_SPLIT

def _mm_body(h_ref, wgt_ref, wgb_ref, ws_ref, bg_ref, bs_ref, p_ref, qs_ref):
    hb = h_ref[...]
    p_ref[...] = jnp.dot(hb, wgt_ref[...], preferred_element_type=jnp.float32)
    q = jnp.dot(hb, wgb_ref[...], preferred_element_type=jnp.float32) + bg_ref[...]
    s = jnp.dot(hb, ws_ref[...], preferred_element_type=jnp.float32) + bs_ref[...]
    qb = lax.bitcast_convert_type(q, jnp.int32) + 0x8000
    sb = lax.bitcast_convert_type(s, jnp.int32) + 0x8000
    qs_ref[...] = jnp.bitwise_or(
        lax.shift_right_logical(qb, 16),
        jnp.bitwise_and(sb, jnp.int32(-65536)))


def _node_tables(h_pad, wgt, wgb, ws, bg, bs):
    blk = N_PAD // 3
    full = lambda shape: pl.BlockSpec(shape, lambda i: (0, 0))
    return pl.pallas_call(
        _mm_body,
        grid=(3,),
        in_specs=[
            pl.BlockSpec((blk, D), lambda i: (i, 0)),
            full((D, D)), full((D, D)), full((D, D)),
            full((1, D)), full((1, D)),
        ],
        out_specs=[
            pl.BlockSpec((blk, D), lambda i: (i, 0)),
            pl.BlockSpec((blk, D), lambda i: (i, 0)),
        ],
        out_shape=[
            jax.ShapeDtypeStruct((N_PAD, D), jnp.float32),
            jax.ShapeDtypeStruct((N_PAD, D), jnp.int32),
        ],
    )(h_pad, wgt, wgb, ws, bg, bs)


def _sc_edge_body(p_hbm, qs_hbm, rec_hbm, send_hbm, out_hbm,
                  rec_v, send_v, buf_p0, buf_p1, buf_qs0, buf_qs1, buf_m,
                  acc, sem_p0, sem_p1, sem_qs0, sem_qs1):
    c = lax.axis_index("c")
    s = lax.axis_index("s")
    chunk_base = jnp.where(c == 0, s * CHUNKS_C0, CORE1_BASE + s * CHUNKS_C1)
    n_groups = jnp.where(c == 0, CHUNKS_C0 // GROUP, CHUNKS_C1 // GROUP)
    buf_p = (buf_p0, buf_p1)
    buf_qs = (buf_qs0, buf_qs1)
    sem_p = (sem_p0, sem_p1)
    sem_qs = (sem_qs0, sem_qs1)

    zero16 = jnp.zeros((L,), jnp.float32)

    def _zrow(i, _):
        for k in range(D // L):
            buf_m[i, pl.ds(k * L, L)] = zero16
        return 0

    lax.fori_loop(0, CHUNK, _zrow, 0)
    base_row = s * ROW_SPLIT

    def _zero_slice(row0, nrows):
        for off in range(0, nrows, CHUNK):
            n = min(CHUNK, nrows - off)
            pltpu.sync_copy(buf_m.at[pl.ds(0, n)],
                            acc.at[pl.ds(row0 + off, n)])

    @pl.when(s < NS - 1)
    def _():
        _zero_slice(base_row, ROW_SPLIT)

    @pl.when(s == NS - 1)
    def _():
        _zero_slice(base_row, ROW_LAST)

    plsc.subcore_barrier()

    def _gate(b):
        p_ref, qs_ref = buf_p[b], buf_qs[b]

        hi_mask = jnp.full((L,), -65536, jnp.int32)

        @plsc.parallel_loop(0, CHUNK * (D // L), unroll=8)
        def _chunk_body(it):
            e = it // (D // L)
            col = (it % (D // L)) * L
            sl = pl.ds(col, L)
            xqs = qs_ref[e, sl]
            q = lax.bitcast_convert_type(lax.shift_left(xqs, 16),
                                         jnp.float32)
            sv = lax.bitcast_convert_type(lax.bitwise_and(xqs, hi_mask),
                                          jnp.float32)
            eta = 1.0 / (1.0 + jnp.exp(-(p_ref[e, sl] + q)))
            buf_m[e, sl] = eta * sv

    def _start_gather(jj, b):
        cp_p = pltpu.async_copy(p_hbm.at[rec_v.at[jj]], buf_p[b], sem_p[b])
        cp_qs = pltpu.async_copy(qs_hbm.at[send_v.at[jj]], buf_qs[b],
                                 sem_qs[b])
        return cp_p, cp_qs

    def _group(g, _):
        slab = chunk_base + g * GROUP
        pltpu.sync_copy(rec_hbm.at[pl.ds(slab, GROUP)], rec_v)
        pltpu.sync_copy(send_hbm.at[pl.ds(slab, GROUP)], send_v)

        cps = _start_gather(0, 0)
        for jj in range(GROUP):
            b = jj % NBUF
            cp_p, cp_qs = cps
            cp_p.wait()
            cp_qs.wait()
            if jj + 1 < GROUP:
                cps = _start_gather(jj + 1, (jj + 1) % NBUF)
            _gate(b)
            pltpu.sync_copy(buf_m, acc.at[rec_v.at[jj]], add=True)
        return 0

    lax.fori_loop(0, n_groups, _group, 0)
    plsc.subcore_barrier()

    @pl.when(s < NS - 1)
    def _():
        pltpu.sync_copy(acc.at[pl.ds(base_row, ROW_SPLIT)],
                        out_hbm.at[c, pl.ds(base_row, ROW_SPLIT)])

    @pl.when(s == NS - 1)
    def _():
        pltpu.sync_copy(acc.at[pl.ds(base_row, ROW_LAST)],
                        out_hbm.at[c, pl.ds(base_row, ROW_LAST)])


_sc_edges = functools.partial(
    pl.kernel,
    mesh=plsc.VectorSubcoreMesh(core_axis_name="c", subcore_axis_name="s"),
    out_type=jax.ShapeDtypeStruct((NC, N_PAD, D), jnp.float32),
    scratch_types=[
        pltpu.VMEM((GROUP, CHUNK), jnp.int32),
        pltpu.VMEM((GROUP, CHUNK), jnp.int32),
        pltpu.VMEM((CHUNK, D), jnp.float32),
        pltpu.VMEM((CHUNK, D), jnp.float32),
        pltpu.VMEM((CHUNK, D), jnp.int32),
        pltpu.VMEM((CHUNK, D), jnp.int32),
        pltpu.VMEM((CHUNK, D), jnp.float32),
        pltpu.VMEM_SHARED((N_PAD, D), jnp.float32),
        pltpu.SemaphoreType.DMA,
        pltpu.SemaphoreType.DMA,
        pltpu.SemaphoreType.DMA,
        pltpu.SemaphoreType.DMA,
    ],
)(_sc_edge_body)


def _fin_body(h_ref, wr_ref, br_ref, agg_ref, out_ref):
    out_ref[...] = (
        jnp.dot(h_ref[...], wr_ref[...], preferred_element_type=jnp.float32)
        + br_ref[...] + agg_ref[0] + agg_ref[1]
    )


def _finalize(h, wr, br, agg):
    blk = N // 5
    return pl.pallas_call(
        _fin_body,
        grid=(5,),
        in_specs=[
            pl.BlockSpec((blk, D), lambda i: (i, 0)),
            pl.BlockSpec((D, D), lambda i: (0, 0)),
            pl.BlockSpec((1, D), lambda i: (0, 0)),
            pl.BlockSpec((NC, blk, D), lambda i: (0, i, 0)),
        ],
        out_specs=pl.BlockSpec((blk, D), lambda i: (i, 0)),
        out_shape=jax.ShapeDtypeStruct((N, D), jnp.float32),
    )(h, wr, br, agg)


@jax.jit
def kernel(h, edge_index, Wg, bg, Ws, bs, Wr, br):
    h_pad = jnp.concatenate(
        [h, jnp.zeros((N_PAD - N, D), jnp.float32)], axis=0)

    p_tab, qs_tab = _node_tables(
        h_pad, Wg[:D], Wg[D:], Ws,
        bg.reshape(1, D), bs.reshape(1, D))

    ei = edge_index.astype(jnp.int32)
    pad = jnp.full((E_PAD - E,), N, jnp.int32)
    rec = jnp.concatenate([ei[1], pad]).reshape(TOTAL_CHUNKS, CHUNK)
    send = jnp.concatenate([ei[0], pad]).reshape(TOTAL_CHUNKS, CHUNK)

    agg = _sc_edges(p_tab, qs_tab, rec, send)

    return _finalize(h, Wr, br.reshape(1, D), agg)


# speedup vs baseline: 1.0339x; 1.0339x over previous
import functools

import jax
import jax.numpy as jnp
from jax import lax
from jax.experimental import pallas as pl
from jax.experimental.pallas import tpu as pltpu
from jax.experimental.pallas import tpu_sc as plsc

N, E, D = 10000, 320000, 128

_INFO = plsc.get_sparse_core_info()
NC, NS, L = _INFO.num_cores, _INFO.num_subcores, _INFO.num_lanes
NW = NC * NS

CHUNK = 64
GROUP = 8
CHUNKS_C0 = 256
CHUNKS_C1 = 64
CORE1_BASE = NS * CHUNKS_C0
TOTAL_CHUNKS = NS * (CHUNKS_C0 + CHUNKS_C1)
E_PAD = TOTAL_CHUNKS * CHUNK
NBUF = 2

N_PAD = 10008
ROW_SPLIT = 632
ROW_LAST = N_PAD - (NS - 1) * ROW_SPLIT

def _mm_body(h_ref, wgt_ref, wgb_ref, ws_ref, bg_ref, bs_ref, p_ref, qs_ref):
    hb = h_ref[...]
    p_ref[...] = jnp.dot(hb, wgt_ref[...], preferred_element_type=jnp.float32)
    q = jnp.dot(hb, wgb_ref[...], preferred_element_type=jnp.float32) + bg_ref[...]
    s = jnp.dot(hb, ws_ref[...], preferred_element_type=jnp.float32) + bs_ref[...]
    qb = lax.bitcast_convert_type(q, jnp.int32) + 0x8000
    sb = lax.bitcast_convert_type(s, jnp.int32) + 0x8000
    qs_ref[...] = jnp.bitwise_or(
        lax.shift_right_logical(qb, 16),
        jnp.bitwise_and(sb, jnp.int32(-65536)))


def _node_tables(h_pad, wgt, wgb, ws, bg, bs):
    blk = N_PAD // 3
    full = lambda shape: pl.BlockSpec(shape, lambda i: (0, 0))
    return pl.pallas_call(
        _mm_body,
        grid=(3,),
        in_specs=[
            pl.BlockSpec((blk, D), lambda i: (i, 0)),
            full((D, D)), full((D, D)), full((D, D)),
            full((1, D)), full((1, D)),
        ],
        out_specs=[
            pl.BlockSpec((blk, D), lambda i: (i, 0)),
            pl.BlockSpec((blk, D), lambda i: (i, 0)),
        ],
        out_shape=[
            jax.ShapeDtypeStruct((N_PAD, D), jnp.float32),
            jax.ShapeDtypeStruct((N_PAD, D), jnp.int32),
        ],
    )(h_pad, wgt, wgb, ws, bg, bs)


def _sc_edge_body(p_hbm, qs_hbm, rec_hbm, send_hbm, out_hbm,
                  rec_v, send_v, buf_p0, buf_p1, buf_qs0, buf_qs1, buf_m,
                  acc, sem_p0, sem_p1, sem_qs0, sem_qs1):
    c = lax.axis_index("c")
    s = lax.axis_index("s")
    chunk_base = jnp.where(c == 0, s * CHUNKS_C0, CORE1_BASE + s * CHUNKS_C1)
    n_groups = jnp.where(c == 0, CHUNKS_C0 // GROUP, CHUNKS_C1 // GROUP)
    buf_p = (buf_p0, buf_p1)
    buf_qs = (buf_qs0, buf_qs1)
    sem_p = (sem_p0, sem_p1)
    sem_qs = (sem_qs0, sem_qs1)

    zero16 = jnp.zeros((L,), jnp.float32)

    def _zrow(i, _):
        for k in range(D // L):
            buf_m[i, pl.ds(k * L, L)] = zero16
        return 0

    lax.fori_loop(0, CHUNK, _zrow, 0)
    base_row = s * ROW_SPLIT

    def _zero_slice(row0, nrows):
        for off in range(0, nrows, CHUNK):
            n = min(CHUNK, nrows - off)
            pltpu.sync_copy(buf_m.at[pl.ds(0, n)],
                            acc.at[pl.ds(row0 + off, n)])

    @pl.when(s < NS - 1)
    def _():
        _zero_slice(base_row, ROW_SPLIT)

    @pl.when(s == NS - 1)
    def _():
        _zero_slice(base_row, ROW_LAST)

    plsc.subcore_barrier()

    def _gate(b):
        p_ref, qs_ref = buf_p[b], buf_qs[b]

        hi_mask = jnp.full((L,), -65536, jnp.int32)

        @plsc.parallel_loop(0, CHUNK * (D // L), unroll=8)
        def _chunk_body(it):
            e = it // (D // L)
            col = (it % (D // L)) * L
            sl = pl.ds(col, L)
            xqs = qs_ref[e, sl]
            q = lax.bitcast_convert_type(lax.shift_left(xqs, 16),
                                         jnp.float32)
            sv = lax.bitcast_convert_type(lax.bitwise_and(xqs, hi_mask),
                                          jnp.float32)
            eta = 1.0 / (1.0 + jnp.exp(-(p_ref[e, sl] + q)))
            buf_m[e, sl] = eta * sv

    def _start_gather(jj, b):
        cp_p = pltpu.async_copy(p_hbm.at[rec_v.at[jj]], buf_p[b], sem_p[b])
        cp_qs = pltpu.async_copy(qs_hbm.at[send_v.at[jj]], buf_qs[b],
                                 sem_qs[b])
        return cp_p, cp_qs

    def _group(g, _):
        slab = chunk_base + g * GROUP
        pltpu.sync_copy(rec_hbm.at[pl.ds(slab, GROUP)], rec_v)
        pltpu.sync_copy(send_hbm.at[pl.ds(slab, GROUP)], send_v)

        cps = _start_gather(0, 0)
        for jj in range(GROUP):
            b = jj % NBUF
            cp_p, cp_qs = cps
            cp_p.wait()
            cp_qs.wait()
            if jj + 1 < GROUP:
                cps = _start_gather(jj + 1, (jj + 1) % NBUF)
            _gate(b)
            pltpu.sync_copy(buf_m, acc.at[rec_v.at[jj]], add=True)
        return 0

    lax.fori_loop(0, n_groups, _group, 0)
    plsc.subcore_barrier()

    @pl.when(s < NS - 1)
    def _():
        pltpu.sync_copy(acc.at[pl.ds(base_row, ROW_SPLIT)],
                        out_hbm.at[c, pl.ds(base_row, ROW_SPLIT)])

    @pl.when(s == NS - 1)
    def _():
        pltpu.sync_copy(acc.at[pl.ds(base_row, ROW_LAST)],
                        out_hbm.at[c, pl.ds(base_row, ROW_LAST)])


_sc_edges = functools.partial(
    pl.kernel,
    mesh=plsc.VectorSubcoreMesh(core_axis_name="c", subcore_axis_name="s"),
    out_type=jax.ShapeDtypeStruct((NC, N_PAD, D), jnp.float32),
    scratch_types=[
        pltpu.VMEM((GROUP, CHUNK), jnp.int32),
        pltpu.VMEM((GROUP, CHUNK), jnp.int32),
        pltpu.VMEM((CHUNK, D), jnp.float32),
        pltpu.VMEM((CHUNK, D), jnp.float32),
        pltpu.VMEM((CHUNK, D), jnp.int32),
        pltpu.VMEM((CHUNK, D), jnp.int32),
        pltpu.VMEM((CHUNK, D), jnp.float32),
        pltpu.VMEM_SHARED((N_PAD, D), jnp.float32),
        pltpu.SemaphoreType.DMA,
        pltpu.SemaphoreType.DMA,
        pltpu.SemaphoreType.DMA,
        pltpu.SemaphoreType.DMA,
    ],
)(_sc_edge_body)


def _fin_body(h_ref, wr_ref, br_ref, agg_ref, out_ref):
    out_ref[...] = (
        jnp.dot(h_ref[...], wr_ref[...], preferred_element_type=jnp.float32)
        + br_ref[...] + agg_ref[0] + agg_ref[1]
    )


def _finalize(h, wr, br, agg):
    blk = N // 5
    return pl.pallas_call(
        _fin_body,
        grid=(5,),
        in_specs=[
            pl.BlockSpec((blk, D), lambda i: (i, 0)),
            pl.BlockSpec((D, D), lambda i: (0, 0)),
            pl.BlockSpec((1, D), lambda i: (0, 0)),
            pl.BlockSpec((NC, blk, D), lambda i: (0, i, 0)),
        ],
        out_specs=pl.BlockSpec((blk, D), lambda i: (i, 0)),
        out_shape=jax.ShapeDtypeStruct((N, D), jnp.float32),
    )(h, wr, br, agg)


@jax.jit
def kernel(h, edge_index, Wg, bg, Ws, bs, Wr, br):
    h_pad = jnp.concatenate(
        [h, jnp.zeros((N_PAD - N, D), jnp.float32)], axis=0)

    p_tab, qs_tab = _node_tables(
        h_pad, Wg[:D], Wg[D:], Ws,
        bg.reshape(1, D), bs.reshape(1, D))

    ei = edge_index.astype(jnp.int32)
    pad = jnp.full((E_PAD - E,), N, jnp.int32)
    rec = jnp.concatenate([ei[1], pad]).reshape(TOTAL_CHUNKS, CHUNK)
    send = jnp.concatenate([ei[0], pad]).reshape(TOTAL_CHUNKS, CHUNK)

    agg = _sc_edges(p_tab, qs_tab, rec, send)

    return _finalize(h, Wr, br.reshape(1, D), agg)
